# Initial kernel scaffold; baseline (speedup 1.0000x reference)
import jax
import jax.numpy as jnp
from jax.experimental import pallas as pl


def kernel(cat_idx, cont, tables, W1, b1, W2, b2, W3, b3, Wf, bf):
    raise NotImplementedError("write your pallas kernel here")


# speedup vs baseline: 7.9270x; 7.9270x over previous
import functools

import numpy as np
import jax
import jax.numpy as jnp
from jax import lax
from jax.experimental import pallas as pl
from jax.experimental.pallas import tpu as pltpu
from jax.experimental.pallas import tpu_sc as plsc


def _sc_geometry():
    try:
        info = plsc.get_sparse_core_info()
        return info.num_cores, info.num_subcores
    except Exception:
        return 2, 16


CH = 128
GROUP = 8


@functools.lru_cache(maxsize=None)
def _make_gather(FV, D, NROW, NC, NS):
    NW = NC * NS
    assert NROW % (NW * GROUP) == 0
    rows_pw = NROW // NW
    groups = rows_pw // GROUP
    mesh = plsc.VectorSubcoreMesh(core_axis_name="c", subcore_axis_name="s")

    def body(tab_hbm, idx_hbm, out_hbm, idx_v, buf, sem):
        wid = lax.axis_index("s") * NC + lax.axis_index("c")
        r0 = wid * rows_pw
        pltpu.sync_copy(idx_hbm.at[pl.ds(r0, rows_pw)], idx_v)

        @pl.loop(0, groups)
        def _group(g):
            cps = []
            for u in range(GROUP):
                cps.append(pltpu.async_copy(
                    tab_hbm.at[idx_v.at[g * GROUP + u]],
                    buf.at[pl.ds(u * CH, CH)], sem))
            for c in cps:
                c.wait()
            out_r = (r0 + g * GROUP) * CH
            pltpu.sync_copy(buf, out_hbm.at[pl.ds(out_r, GROUP * CH)])

    return pl.kernel(
        body,
        out_type=jax.ShapeDtypeStruct((NROW * CH, D), jnp.float32),
        mesh=mesh,
        compiler_params=pltpu.CompilerParams(use_tc_tiling_on_sc=False),
        scratch_types=[
            pltpu.VMEM((rows_pw, CH), jnp.int32),
            pltpu.VMEM((GROUP * CH, D), jnp.float32),
            pltpu.SemaphoreType.DMA,
        ],
    )


def _mlp_body(x_ref, c_ref, w1a, w1b, b1, w2, b2, w3, b3, wf, bf, sp, o_ref):
    x = x_ref[...]
    f32 = jnp.float32
    s = jnp.dot(x, sp[...], preferred_element_type=f32)
    fm = 0.5 * (jnp.sum(s * s, axis=1) - jnp.sum(x * x, axis=1))
    h = jnp.dot(x, w1a[...], preferred_element_type=f32)
    h = h + jnp.dot(c_ref[...], w1b[...], preferred_element_type=f32)
    h = jnp.maximum(h + b1[...], 0.0)
    h = jnp.maximum(jnp.dot(h, w2[...], preferred_element_type=f32) + b2[...], 0.0)
    h = jnp.maximum(jnp.dot(h, w3[...], preferred_element_type=f32) + b3[...], 0.0)
    deep = jnp.sum(h * wf[...], axis=1) + bf[0, 0]
    z = fm + deep
    o_ref[...] = 1.0 / (1.0 + jnp.exp(-z))


@functools.lru_cache(maxsize=None)
def _make_mlp(B, E, C, H1, H2, H3, Bb, interpret=False):
    grid = (B // Bb,)
    full = lambda r, c: pl.BlockSpec((r, c), lambda i: (0, 0))
    return pl.pallas_call(
        _mlp_body,
        grid=grid,
        in_specs=[
            pl.BlockSpec((Bb, E), lambda i: (i, 0)),
            pl.BlockSpec((Bb, C), lambda i: (i, 0)),
            full(E, H1), full(C, H1), full(1, H1),
            full(H1, H2), full(1, H2),
            full(H2, H3), full(1, H3),
            full(1, H3), full(1, 1),
            full(E, 128),
        ],
        out_specs=pl.BlockSpec((Bb,), lambda i: (i,)),
        out_shape=jax.ShapeDtypeStruct((B,), jnp.float32),
        compiler_params=pltpu.CompilerParams(
            dimension_semantics=("arbitrary",)),
        interpret=interpret,
    )


def kernel(cat_idx, cont, tables, W1, b1, W2, b2, W3, b3, Wf, bf):
    B, F = cat_idx.shape
    _, V, D = tables.shape
    C = cont.shape[1]
    E = F * D
    H1, H2, H3 = W1.shape[1], W2.shape[1], W3.shape[1]

    tab = tables.reshape(F * V, D)
    offs = (jnp.arange(F, dtype=jnp.int32) * V)[None, :]
    fidx = (cat_idx.astype(jnp.int32) + offs).reshape(-1, CH)
    NC, NS = _sc_geometry()
    emb = _make_gather(F * V, D, fidx.shape[0], NC, NS)(tab, fidx)
    embf = emb.reshape(B, E)

    sp = np.zeros((E, 128), np.float32)
    for d in range(D):
        sp[np.arange(F) * D + d, d] = 1.0
    sp = jnp.asarray(sp)

    out = _make_mlp(B, E, C, H1, H2, H3, 512)(
        embf, cont,
        W1[:E], W1[E:], b1.reshape(1, H1),
        W2, b2.reshape(1, H2),
        W3, b3.reshape(1, H3),
        Wf.reshape(1, H3), bf.reshape(1, 1),
        sp)
    return out.reshape(B, 1)
